# baseline (speedup 1.0000x reference)
import functools

import jax
import jax.numpy as jnp
from jax import lax
from jax.experimental import pallas as pl
from jax.experimental.pallas import tpu as pltpu
from jax.experimental.pallas import tpu_sc as plsc

B = 32
DF = 768
M = 14
MM = 196
C = 3000
V = 20000
K = 10
DE = 1024
DT = 1024
DJ = 1024
DW = 300
DWP = 384
NC, NS = 2, 16
CHUNKS = 188
GC = 16
NG = 12


def _sc_body(logits_hbm, itab_hbm, wemb_hbm,
             tki_hbm, tkc_hbm, cls0_hbm,
             row_v, itab_v, tkc_v, widx_v, rows_v, tki_v, cls0_v, gmax_v,
             scrv_v, scri_v, sem, sem2):
    b = lax.axis_index("s") * NC + lax.axis_index("c")
    itab_cp = pltpu.async_copy(itab_hbm, itab_v, sem2)
    pltpu.sync_copy(logits_hbm.at[pl.ds(b * C, C)], row_v.at[pl.ds(0, C)])

    lane = lax.iota(jnp.int32, 16)
    neg = jnp.full((16,), -jnp.inf, jnp.float32)

    def bfly_argmax(bv, bi):
        for sh in (8, 4, 2, 1):
            scrv_v[...] = bv
            scri_v[...] = bi
            pv = plsc.load_gather(scrv_v, [lane ^ sh])
            pi = plsc.load_gather(scri_v, [lane ^ sh])
            m = (pv > bv) | ((pv == bv) & (pi < bi))
            bv = jnp.where(m, pv, bv)
            bi = jnp.where(m, pi, bi)
        return bv, bi

    tail = row_v[pl.ds(C - 8, 16)]
    row_v[pl.ds(C - 8, 16)] = jnp.where(lane < 8, tail, neg)
    for ch in range(CHUNKS, NG * GC):
        row_v[pl.ds(ch * 16, 16)] = neg

    for g in range(NG):
        def gbody(j, gm):
            off = pl.multiple_of(g * GC * 16 + j * 16, 16)
            return jnp.maximum(gm, row_v[pl.ds(off, 16)])
        gmax_v[pl.ds(g * 16, 16)] = lax.fori_loop(0, GC, gbody, neg)

    tki = jnp.zeros((16,), jnp.int32)
    for t in range(K):
        def sbody(g, carry):
            bv, bg = carry
            off = pl.multiple_of(g * 16, 16)
            gm = gmax_v[pl.ds(off, 16)]
            m = gm > bv
            return jnp.where(m, gm, bv), jnp.where(m, g, bg)
        bv, bg = lax.fori_loop(0, NG, sbody,
                               (neg, jnp.zeros((16,), jnp.int32)))
        _, bg = bfly_argmax(bv, bg)
        g = bg[0]

        def rbody(j, carry):
            bv, bi = carry
            off = pl.multiple_of(g * (GC * 16), 16) + pl.multiple_of(j * 16, 16)
            vv = row_v[pl.ds(off, 16)]
            m = vv > bv
            return jnp.where(m, vv, bv), jnp.where(m, off + lane, bi)
        bv, bi = lax.fori_loop(0, GC, rbody,
                               (neg, jnp.zeros((16,), jnp.int32)))
        bv, bi = bfly_argmax(bv, bi)
        tki = jnp.where(lane == t, bi, tki)
        plsc.store_scatter(row_v, [bi], neg, mask=lane == 0)

        def ubody(j, gm):
            off = pl.multiple_of(g * (GC * 16), 16) + pl.multiple_of(j * 16, 16)
            return jnp.maximum(gm, row_v[pl.ds(off, 16)])
        gmax_v[pl.ds(pl.multiple_of(g * 16, 16), 16)] = (
            lax.fori_loop(0, GC, ubody, neg))
    tki_v[...] = tki
    itab_cp.wait()

    base = tki * 4
    for c in range(4):
        w = plsc.load_gather(itab_v, [base + c])
        plsc.store_scatter(tkc_v, [lane * 4 + c], w)
        plsc.store_scatter(widx_v, [lane * 4 + c], w, mask=lane < K)

    tki_cp = pltpu.async_copy(tki_v, tki_hbm.at[pl.ds(b * 16, 16)], sem2)
    tkc_cp = pltpu.async_copy(tkc_v, tkc_hbm.at[pl.ds(b * 64, 64)], sem2)

    pltpu.async_copy(wemb_hbm.at[widx_v], rows_v, sem).wait()

    for k in range(K):
        for ch in range(DWP // 16):
            s = (rows_v[4 * k + 0, pl.ds(ch * 16, 16)]
                 + rows_v[4 * k + 1, pl.ds(ch * 16, 16)]
                 + rows_v[4 * k + 2, pl.ds(ch * 16, 16)]
                 + rows_v[4 * k + 3, pl.ds(ch * 16, 16)]) * 0.25
            cls0_v[k, pl.ds(ch * 16, 16)] = s
    pltpu.sync_copy(cls0_v, cls0_hbm.at[b])
    tki_cp.wait()
    tkc_cp.wait()


@functools.cache
def _get_sc_call():
  return pl.kernel(
    _sc_body,
    out_type=(
        jax.ShapeDtypeStruct((B * 16,), jnp.int32),
        jax.ShapeDtypeStruct((B * 64,), jnp.int32),
        jax.ShapeDtypeStruct((B, 16, DWP), jnp.float32),
    ),
    mesh=plsc.VectorSubcoreMesh(core_axis_name="c", subcore_axis_name="s",
                                num_cores=NC, num_subcores=NS),
    compiler_params=pltpu.CompilerParams(needs_layout_passes=False),
    scratch_types=[
        pltpu.VMEM((NG * GC * 16,), jnp.float32),
        pltpu.VMEM((C * 4,), jnp.int32),
        pltpu.VMEM((64,), jnp.int32),
        pltpu.VMEM((40,), jnp.int32),
        pltpu.VMEM((40, DWP), jnp.float32),
        pltpu.VMEM((16,), jnp.int32),
        pltpu.VMEM((16, DWP), jnp.float32),
        pltpu.VMEM((NG * 16,), jnp.float32),
        pltpu.VMEM((16,), jnp.float32),
        pltpu.VMEM((16,), jnp.int32),
        pltpu.SemaphoreType.DMA,
        pltpu.SemaphoreType.DMA,
    ],
  )


def _pad_body(x_ref, o_ref):
    o_ref[...] = jnp.concatenate(
        [x_ref[...], jnp.zeros((x_ref.shape[0], DWP - DW), jnp.float32)],
        axis=1)


_kpad = pl.pallas_call(
    _pad_body,
    grid=(V // 2000,),
    in_specs=[pl.BlockSpec((2000, DW), lambda i: (i, 0))],
    out_specs=pl.BlockSpec((2000, DWP), lambda i: (i, 0)),
    out_shape=jax.ShapeDtypeStruct((V, DWP), jnp.float32),
)


def _mega_body(cls0_ref, wcls_ref, bcls_ref, w1_ref, b1_ref,
               x_ref, w2_ref, b2_ref, w3_ref, b3_ref, wc_ref, bc_ref,
               tki_ref, attw_ref, sfx_ref, um_ref, ce_s, bap_s):
    i = pl.program_id(0)

    @pl.when(i == 0)
    def _k1():
        h = jnp.tanh(jnp.dot(cls0_ref[...], wcls_ref[...],
                             preferred_element_type=jnp.float32)
                     + bcls_ref[...])
        ce_s[...] = jnp.maximum(
            jnp.dot(h, w1_ref[...], preferred_element_type=jnp.float32)
            + b1_ref[...], 0.0)

    x = x_ref[...]
    vT = jnp.maximum(
        lax.dot_general(w2_ref[...], x, (((0,), (0,)), ((), ())),
                        preferred_element_type=jnp.float32)
        + b2_ref[...], 0.0)
    ce = ce_s[pl.ds(i * 16, 16)][0:K]
    sfx = lax.dot_general(ce, vT, (((1,), (0,)), ((), ())),
                          preferred_element_type=jnp.float32)
    m = jnp.max(sfx, axis=1, keepdims=True)
    e = jnp.exp(sfx - m)
    attw = e / jnp.sum(e, axis=1, keepdims=True)
    sfx_ref[0] = sfx.T
    attw_ref[0] = attw.T
    t = lax.dot_general(attw, vT, (((1,), (1,)), ((), ())),
                        preferred_element_type=jnp.float32)
    bap_s[pl.ds(i, 1)] = jnp.sum(t * ce, axis=0, keepdims=True)

    @pl.when(i == B - 1)
    def _k3():
        h = jnp.maximum(
            jnp.dot(bap_s[...], w3_ref[...],
                    preferred_element_type=jnp.float32) + b3_ref[...], 0.0)
        lg = jnp.dot(h, wc_ref[...],
                     preferred_element_type=jnp.float32) + bc_ref[...]
        cid = lax.broadcasted_iota(jnp.int32, (B, C), 1)
        tki = tki_ref[...]
        um = jnp.zeros((B, C), jnp.float32)
        for k in range(K):
            um = um + jnp.where(cid == tki[:, k:k + 1], lg[:, k:k + 1], 0.0)
        um_ref[...] = um


_zero3 = lambda i: (0, 0, 0)
_zero2 = lambda i: (0, 0)
_mega = pl.pallas_call(
    _mega_body,
    grid=(B,),
    in_specs=[
        pl.BlockSpec((B * 16, DWP), _zero2),
        pl.BlockSpec((DWP, DE), _zero2),
        pl.BlockSpec((1, DE), _zero2),
        pl.BlockSpec((DE, DT), _zero2),
        pl.BlockSpec((1, DT), _zero2),
        pl.BlockSpec((DF, MM), lambda i: (i, 0)),
        pl.BlockSpec((DF, DT), _zero2),
        pl.BlockSpec((DT, 1), _zero2),
        pl.BlockSpec((DT, DJ), _zero2),
        pl.BlockSpec((1, DJ), _zero2),
        pl.BlockSpec((DJ, K), _zero2),
        pl.BlockSpec((1, K), _zero2),
        pl.BlockSpec((B, K), _zero2),
    ],
    out_specs=[
        pl.BlockSpec((1, MM, K), lambda i: (i, 0, 0)),
        pl.BlockSpec((1, MM, K), lambda i: (i, 0, 0)),
        pl.BlockSpec((B, C), _zero2),
    ],
    out_shape=[
        jax.ShapeDtypeStruct((B, MM, K), jnp.float32),
        jax.ShapeDtypeStruct((B, MM, K), jnp.float32),
        jax.ShapeDtypeStruct((B, C), jnp.float32),
    ],
    scratch_shapes=[
        pltpu.VMEM((B * 16, DT), jnp.float32),
        pltpu.VMEM((B, DT), jnp.float32),
    ],
)


def kernel(ftm, logits, inputs, labels, indices_table, word_emb,
           W_cls, b_cls, W1, b1, W2, b2, W3, b3, Wc, bc):
    itab = indices_table.reshape(-1)
    wemb = _kpad(word_emb)
    tki_p, tkc_p, cls0 = _get_sc_call()(logits.reshape(-1), itab, wemb)
    topk_idx = tki_p.reshape(B, 16)[:, :K]
    topk_cls = tkc_p.reshape(B, 16, 4)[:, :K, :]
    wcls_p = jnp.pad(W_cls, ((0, DWP - DW), (0, 0)))
    att_w, att_sfx, um = _mega(cls0.reshape(B * 16, DWP), wcls_p,
                               b_cls[None], W1, b1[None],
                               ftm.reshape(B * DF, MM), W2, b2[:, None],
                               W3, b3[None], Wc, bc[None], topk_idx)
    return um, att_w, topk_cls, att_sfx, topk_idx, labels


# speedup vs baseline: 1.0856x; 1.0856x over previous
import functools

import jax
import jax.numpy as jnp
from jax import lax
from jax.experimental import pallas as pl
from jax.experimental.pallas import tpu as pltpu
from jax.experimental.pallas import tpu_sc as plsc

B = 32
DF = 768
M = 14
MM = 196
C = 3000
V = 20000
K = 10
DE = 1024
DT = 1024
DJ = 1024
DW = 300
DWP = 384
NC, NS = 2, 16
CHUNKS = 188
GC = 16
NG = 12


def _sc_body(logits_hbm, itab_hbm, wemb_hbm,
             tki_hbm, tkc_hbm, cls0_hbm,
             row_v, itab_v, tkc_v, widx_v, rows_v, tki_v, cls0_v, gmax_v,
             scrv_v, scri_v, sem, sem2):
    b = lax.axis_index("s") * NC + lax.axis_index("c")
    itab_cp = pltpu.async_copy(itab_hbm, itab_v, sem2)
    pltpu.sync_copy(logits_hbm.at[pl.ds(b * C, C)], row_v.at[pl.ds(0, C)])

    lane = lax.iota(jnp.int32, 16)
    neg = jnp.full((16,), -jnp.inf, jnp.float32)

    def bfly_argmax(bv, bi):
        for sh in (8, 4, 2, 1):
            scrv_v[...] = bv
            scri_v[...] = bi
            pv = plsc.load_gather(scrv_v, [lane ^ sh])
            pi = plsc.load_gather(scri_v, [lane ^ sh])
            m = (pv > bv) | ((pv == bv) & (pi < bi))
            bv = jnp.where(m, pv, bv)
            bi = jnp.where(m, pi, bi)
        return bv, bi

    tail = row_v[pl.ds(C - 8, 16)]
    row_v[pl.ds(C - 8, 16)] = jnp.where(lane < 8, tail, neg)
    for ch in range(CHUNKS, NG * GC):
        row_v[pl.ds(ch * 16, 16)] = neg

    for g in range(NG):
        def gbody(j, gm):
            off = pl.multiple_of(g * GC * 16 + j * 16, 16)
            return jnp.maximum(gm, row_v[pl.ds(off, 16)])
        gmax_v[pl.ds(g * 16, 16)] = lax.fori_loop(0, GC, gbody, neg)

    tki = jnp.zeros((16,), jnp.int32)
    for t in range(K):
        def sbody(g, carry):
            bv, bg = carry
            off = pl.multiple_of(g * 16, 16)
            gm = gmax_v[pl.ds(off, 16)]
            m = gm > bv
            return jnp.where(m, gm, bv), jnp.where(m, g, bg)
        bv, bg = lax.fori_loop(0, NG, sbody,
                               (neg, jnp.zeros((16,), jnp.int32)))
        _, bg = bfly_argmax(bv, bg)
        g = bg[0]

        def rbody(j, carry):
            bv, bi = carry
            off = pl.multiple_of(g * (GC * 16), 16) + pl.multiple_of(j * 16, 16)
            vv = row_v[pl.ds(off, 16)]
            m = vv > bv
            return jnp.where(m, vv, bv), jnp.where(m, off + lane, bi)
        bv, bi = lax.fori_loop(0, GC, rbody,
                               (neg, jnp.zeros((16,), jnp.int32)))
        bv, bi = bfly_argmax(bv, bi)
        tki = jnp.where(lane == t, bi, tki)
        plsc.store_scatter(row_v, [bi], neg, mask=lane == 0)

        def ubody(j, gm):
            off = pl.multiple_of(g * (GC * 16), 16) + pl.multiple_of(j * 16, 16)
            return jnp.maximum(gm, row_v[pl.ds(off, 16)])
        gmax_v[pl.ds(pl.multiple_of(g * 16, 16), 16)] = (
            lax.fori_loop(0, GC, ubody, neg))
    tki_v[...] = tki
    itab_cp.wait()

    base = tki * 4
    for c in range(4):
        w = plsc.load_gather(itab_v, [base + c])
        plsc.store_scatter(tkc_v, [lane * 4 + c], w)
        plsc.store_scatter(widx_v, [lane * 4 + c], w, mask=lane < K)

    tki_cp = pltpu.async_copy(tki_v, tki_hbm.at[pl.ds(b * 16, 16)], sem2)
    tkc_cp = pltpu.async_copy(tkc_v, tkc_hbm.at[pl.ds(b * 64, 64)], sem2)

    pltpu.async_copy(wemb_hbm.at[widx_v], rows_v, sem).wait()

    for k in range(K):
        for ch in range(DWP // 16):
            s = (rows_v[4 * k + 0, pl.ds(ch * 16, 16)]
                 + rows_v[4 * k + 1, pl.ds(ch * 16, 16)]
                 + rows_v[4 * k + 2, pl.ds(ch * 16, 16)]
                 + rows_v[4 * k + 3, pl.ds(ch * 16, 16)]) * 0.25
            cls0_v[k, pl.ds(ch * 16, 16)] = s
    pltpu.sync_copy(cls0_v, cls0_hbm.at[b])
    tki_cp.wait()
    tkc_cp.wait()


@functools.cache
def _get_sc_call():
  return pl.kernel(
    _sc_body,
    out_type=(
        jax.ShapeDtypeStruct((B * 16,), jnp.int32),
        jax.ShapeDtypeStruct((B * 64,), jnp.int32),
        jax.ShapeDtypeStruct((B, 16, DWP), jnp.float32),
    ),
    mesh=plsc.VectorSubcoreMesh(core_axis_name="c", subcore_axis_name="s",
                                num_cores=NC, num_subcores=NS),
    compiler_params=pltpu.CompilerParams(needs_layout_passes=False),
    scratch_types=[
        pltpu.VMEM((NG * GC * 16,), jnp.float32),
        pltpu.VMEM((C * 4,), jnp.int32),
        pltpu.VMEM((64,), jnp.int32),
        pltpu.VMEM((40,), jnp.int32),
        pltpu.VMEM((40, DWP), jnp.float32),
        pltpu.VMEM((16,), jnp.int32),
        pltpu.VMEM((16, DWP), jnp.float32),
        pltpu.VMEM((NG * 16,), jnp.float32),
        pltpu.VMEM((16,), jnp.float32),
        pltpu.VMEM((16,), jnp.int32),
        pltpu.SemaphoreType.DMA,
        pltpu.SemaphoreType.DMA,
    ],
  )


def _pad_body(x_ref, o_ref):
    o_ref[...] = jnp.concatenate(
        [x_ref[...], jnp.zeros((x_ref.shape[0], DWP - DW), jnp.float32)],
        axis=1)


_kpad = pl.pallas_call(
    _pad_body,
    grid=(V // 2000,),
    in_specs=[pl.BlockSpec((2000, DW), lambda i: (i, 0))],
    out_specs=pl.BlockSpec((2000, DWP), lambda i: (i, 0)),
    out_shape=jax.ShapeDtypeStruct((V, DWP), jnp.float32),
)


def _mega_body(cls0_ref, wcls_ref, bcls_ref, w1_ref, b1_ref,
               x_ref, w2_ref, b2_ref, w3_ref, b3_ref, wc_ref, bc_ref,
               tki_ref, attw_ref, sfx_ref, um_ref, ce_s, bap_s):
    i = pl.program_id(0)

    @pl.when(i == 0)
    def _k1():
        h = jnp.tanh(jnp.dot(cls0_ref[...], wcls_ref[...],
                             preferred_element_type=jnp.float32)
                     + bcls_ref[...])
        ce_s[...] = jnp.maximum(
            jnp.dot(h, w1_ref[...], preferred_element_type=jnp.float32)
            + b1_ref[...], 0.0)

    x = x_ref[0]
    vT = jnp.maximum(
        lax.dot_general(w2_ref[...], x, (((0,), (0,)), ((), ())),
                        preferred_element_type=jnp.float32)
        + b2_ref[...], 0.0)
    ce = ce_s[pl.ds(i * 16, 16)][0:K]
    sfx = lax.dot_general(ce, vT, (((1,), (0,)), ((), ())),
                          preferred_element_type=jnp.float32)
    m = jnp.max(sfx, axis=1, keepdims=True)
    e = jnp.exp(sfx - m)
    attw = e / jnp.sum(e, axis=1, keepdims=True)
    sfx_ref[0] = sfx.T
    attw_ref[0] = attw.T
    t = lax.dot_general(attw, vT, (((1,), (1,)), ((), ())),
                        preferred_element_type=jnp.float32)
    bap_s[pl.ds(i, 1)] = jnp.sum(t * ce, axis=0, keepdims=True)

    @pl.when(i == B - 1)
    def _k3():
        h = jnp.maximum(
            jnp.dot(bap_s[...], w3_ref[...],
                    preferred_element_type=jnp.float32) + b3_ref[...], 0.0)
        lg = jnp.dot(h, wc_ref[...],
                     preferred_element_type=jnp.float32) + bc_ref[...]
        cid = lax.broadcasted_iota(jnp.int32, (B, C), 1)
        tki = tki_ref[...]
        um = jnp.zeros((B, C), jnp.float32)
        for k in range(K):
            um = um + jnp.where(cid == tki[:, k:k + 1], lg[:, k:k + 1], 0.0)
        um_ref[...] = um


_zero3 = lambda i: (0, 0, 0)
_zero2 = lambda i: (0, 0)
_mega = pl.pallas_call(
    _mega_body,
    grid=(B,),
    in_specs=[
        pl.BlockSpec((B * 16, DWP), _zero2),
        pl.BlockSpec((DWP, DE), _zero2),
        pl.BlockSpec((1, DE), _zero2),
        pl.BlockSpec((DE, DT), _zero2),
        pl.BlockSpec((1, DT), _zero2),
        pl.BlockSpec((1, DF, MM), lambda i: (i, 0, 0)),
        pl.BlockSpec((DF, DT), _zero2),
        pl.BlockSpec((DT, 1), _zero2),
        pl.BlockSpec((DT, DJ), _zero2),
        pl.BlockSpec((1, DJ), _zero2),
        pl.BlockSpec((DJ, K), _zero2),
        pl.BlockSpec((1, K), _zero2),
        pl.BlockSpec((B, K), _zero2),
    ],
    out_specs=[
        pl.BlockSpec((1, MM, K), lambda i: (i, 0, 0)),
        pl.BlockSpec((1, MM, K), lambda i: (i, 0, 0)),
        pl.BlockSpec((B, C), _zero2),
    ],
    out_shape=[
        jax.ShapeDtypeStruct((B, MM, K), jnp.float32),
        jax.ShapeDtypeStruct((B, MM, K), jnp.float32),
        jax.ShapeDtypeStruct((B, C), jnp.float32),
    ],
    scratch_shapes=[
        pltpu.VMEM((B * 16, DT), jnp.float32),
        pltpu.VMEM((B, DT), jnp.float32),
    ],
)


def kernel(ftm, logits, inputs, labels, indices_table, word_emb,
           W_cls, b_cls, W1, b1, W2, b2, W3, b3, Wc, bc):
    itab = indices_table.reshape(-1)
    wemb = _kpad(word_emb)
    tki_p, tkc_p, cls0 = _get_sc_call()(logits.reshape(-1), itab, wemb)
    topk_idx = tki_p.reshape(B, 16)[:, :K]
    topk_cls = tkc_p.reshape(B, 16, 4)[:, :K, :]
    wcls_p = jnp.pad(W_cls, ((0, DWP - DW), (0, 0)))
    att_w, att_sfx, um = _mega(cls0.reshape(B * 16, DWP), wcls_p,
                               b_cls[None], W1, b1[None],
                               ftm.astype(jnp.bfloat16).reshape(B, DF, MM),
                               W2.astype(jnp.bfloat16), b2[:, None],
                               W3, b3[None], Wc, bc[None], topk_idx)
    return um, att_w, topk_cls, att_sfx, topk_idx, labels


# speedup vs baseline: 1.1365x; 1.0469x over previous
import functools

import jax
import jax.numpy as jnp
from jax import lax
from jax.experimental import pallas as pl
from jax.experimental.pallas import tpu as pltpu
from jax.experimental.pallas import tpu_sc as plsc

B = 32
DF = 768
M = 14
MM = 196
C = 3000
V = 20000
K = 10
DE = 1024
DT = 1024
DJ = 1024
DW = 300
DWP = 384
NC, NS = 2, 16
CHUNKS = 188
GC = 16
NG = 12


def _sc_body(logits_hbm, itab_hbm, wemb_hbm,
             tki_hbm, tkc_hbm, cls0_hbm,
             row_v, itab_v, tkc_v, widx_v, rows_v, tki_v, cls0_v, gmax_v,
             scrv_v, scri_v, sem, sem2):
    b = lax.axis_index("s") * NC + lax.axis_index("c")
    itab_cp = pltpu.async_copy(itab_hbm, itab_v, sem2)
    pltpu.sync_copy(logits_hbm.at[pl.ds(b * C, C)], row_v.at[pl.ds(0, C)])

    lane = lax.iota(jnp.int32, 16)
    neg = jnp.full((16,), -jnp.inf, jnp.float32)

    def bfly_argmax(bv, bi):
        for sh in (8, 4, 2, 1):
            scrv_v[...] = bv
            scri_v[...] = bi
            pv = plsc.load_gather(scrv_v, [lane ^ sh])
            pi = plsc.load_gather(scri_v, [lane ^ sh])
            m = (pv > bv) | ((pv == bv) & (pi < bi))
            bv = jnp.where(m, pv, bv)
            bi = jnp.where(m, pi, bi)
        return bv, bi

    tail = row_v[pl.ds(C - 8, 16)]
    row_v[pl.ds(C - 8, 16)] = jnp.where(lane < 8, tail, neg)
    for ch in range(CHUNKS, NG * GC):
        row_v[pl.ds(ch * 16, 16)] = neg

    for g in range(NG):
        def gbody(j, gm):
            off = pl.multiple_of(g * GC * 16 + j * 16, 16)
            return jnp.maximum(gm, row_v[pl.ds(off, 16)])
        gmax_v[pl.ds(g * 16, 16)] = lax.fori_loop(0, GC, gbody, neg)

    tki = jnp.zeros((16,), jnp.int32)
    for t in range(K):
        def sbody(g, carry):
            bv, bg = carry
            off = pl.multiple_of(g * 16, 16)
            gm = gmax_v[pl.ds(off, 16)]
            m = gm > bv
            return jnp.where(m, gm, bv), jnp.where(m, g, bg)
        bv, bg = lax.fori_loop(0, NG, sbody,
                               (neg, jnp.zeros((16,), jnp.int32)))
        _, bg = bfly_argmax(bv, bg)
        g = bg[0]

        def rbody(j, carry):
            bv, bi = carry
            off = pl.multiple_of(g * (GC * 16), 16) + pl.multiple_of(j * 16, 16)
            vv = row_v[pl.ds(off, 16)]
            m = vv > bv
            return jnp.where(m, vv, bv), jnp.where(m, off + lane, bi)
        bv, bi = lax.fori_loop(0, GC, rbody,
                               (neg, jnp.zeros((16,), jnp.int32)))
        bv, bi = bfly_argmax(bv, bi)
        tki = jnp.where(lane == t, bi, tki)
        plsc.store_scatter(row_v, [bi], neg, mask=lane == 0)

        def ubody(j, gm):
            off = pl.multiple_of(g * (GC * 16), 16) + pl.multiple_of(j * 16, 16)
            return jnp.maximum(gm, row_v[pl.ds(off, 16)])
        gmax_v[pl.ds(pl.multiple_of(g * 16, 16), 16)] = (
            lax.fori_loop(0, GC, ubody, neg))
    tki_v[...] = tki
    itab_cp.wait()

    base = tki * 4
    for c in range(4):
        w = plsc.load_gather(itab_v, [base + c])
        plsc.store_scatter(tkc_v, [lane * 4 + c], w)
        plsc.store_scatter(widx_v, [lane * 4 + c], w, mask=lane < K)

    tki_cp = pltpu.async_copy(tki_v, tki_hbm.at[pl.ds(b * 16, 16)], sem2)
    tkc_cp = pltpu.async_copy(tkc_v, tkc_hbm.at[pl.ds(b * 64, 64)], sem2)

    pltpu.async_copy(wemb_hbm.at[widx_v], rows_v, sem).wait()

    for k in range(K):
        for ch in range(DWP // 16):
            s = (rows_v[4 * k + 0, pl.ds(ch * 16, 16)]
                 + rows_v[4 * k + 1, pl.ds(ch * 16, 16)]
                 + rows_v[4 * k + 2, pl.ds(ch * 16, 16)]
                 + rows_v[4 * k + 3, pl.ds(ch * 16, 16)]) * 0.25
            cls0_v[k, pl.ds(ch * 16, 16)] = s
    pltpu.sync_copy(cls0_v, cls0_hbm.at[b])
    tki_cp.wait()
    tkc_cp.wait()


@functools.cache
def _get_sc_call():
  return pl.kernel(
    _sc_body,
    out_type=(
        jax.ShapeDtypeStruct((B * 16,), jnp.int32),
        jax.ShapeDtypeStruct((B * 64,), jnp.int32),
        jax.ShapeDtypeStruct((B, 16, DWP), jnp.float32),
    ),
    mesh=plsc.VectorSubcoreMesh(core_axis_name="c", subcore_axis_name="s",
                                num_cores=NC, num_subcores=NS),
    compiler_params=pltpu.CompilerParams(needs_layout_passes=False),
    scratch_types=[
        pltpu.VMEM((NG * GC * 16,), jnp.float32),
        pltpu.VMEM((C * 4,), jnp.int32),
        pltpu.VMEM((64,), jnp.int32),
        pltpu.VMEM((40,), jnp.int32),
        pltpu.VMEM((40, DWP), jnp.float32),
        pltpu.VMEM((16,), jnp.int32),
        pltpu.VMEM((16, DWP), jnp.float32),
        pltpu.VMEM((NG * 16,), jnp.float32),
        pltpu.VMEM((16,), jnp.float32),
        pltpu.VMEM((16,), jnp.int32),
        pltpu.SemaphoreType.DMA,
        pltpu.SemaphoreType.DMA,
    ],
  )


def _pad_body(x_ref, o_ref):
    o_ref[...] = jnp.concatenate(
        [x_ref[...], jnp.zeros((x_ref.shape[0], DWP - DW), jnp.float32)],
        axis=1)


_kpad = pl.pallas_call(
    _pad_body,
    grid=(V // 2000,),
    in_specs=[pl.BlockSpec((2000, DW), lambda i: (i, 0))],
    out_specs=pl.BlockSpec((2000, DWP), lambda i: (i, 0)),
    out_shape=jax.ShapeDtypeStruct((V, DWP), jnp.float32),
)


def _mega_body(cls0_ref, wcls_ref, bcls_ref, w1_ref, b1_ref,
               x_ref, w2_ref, b2_ref, w3_ref, b3_ref, wc_ref, bc_ref,
               tki_ref, attw_ref, sfx_ref, um_ref, ce_s, bap_s, xbuf, semx):
    i = pl.program_id(0)
    slot = lax.rem(i, 2)
    nxt = lax.rem(i + 1, 2)

    @pl.when(i == 0)
    def _pre():
        pltpu.make_async_copy(x_ref.at[0], xbuf.at[0], semx.at[0]).start()

    @pl.when(i + 1 < B)
    def _next():
        pltpu.make_async_copy(x_ref.at[i + 1], xbuf.at[nxt],
                              semx.at[nxt]).start()

    @pl.when(i == 0)
    def _k1():
        h = jnp.tanh(jnp.dot(cls0_ref[...], wcls_ref[...],
                             preferred_element_type=jnp.float32)
                     + bcls_ref[...])
        ce_s[...] = jnp.maximum(
            jnp.dot(h, w1_ref[...], preferred_element_type=jnp.float32)
            + b1_ref[...], 0.0)

    pltpu.make_async_copy(x_ref.at[i], xbuf.at[slot], semx.at[slot]).wait()
    x = xbuf[pl.ds(slot, 1)][0]
    vT = jnp.maximum(
        lax.dot_general(w2_ref[...], x, (((0,), (0,)), ((), ())),
                        preferred_element_type=jnp.float32)
        + b2_ref[...], 0.0)
    ce = ce_s[pl.ds(i * 16, 16)][0:K]
    sfx = lax.dot_general(ce, vT, (((1,), (0,)), ((), ())),
                          preferred_element_type=jnp.float32)
    m = jnp.max(sfx, axis=1, keepdims=True)
    e = jnp.exp(sfx - m)
    attw = e / jnp.sum(e, axis=1, keepdims=True)
    sfx_ref[0] = sfx.T
    attw_ref[0] = attw.T
    t = lax.dot_general(attw, vT, (((1,), (1,)), ((), ())),
                        preferred_element_type=jnp.float32)
    bap_s[pl.ds(i, 1)] = jnp.sum(t * ce, axis=0, keepdims=True)

    @pl.when(i == B - 1)
    def _k3():
        h = jnp.maximum(
            jnp.dot(bap_s[...], w3_ref[...],
                    preferred_element_type=jnp.float32) + b3_ref[...], 0.0)
        lg = jnp.dot(h, wc_ref[...],
                     preferred_element_type=jnp.float32) + bc_ref[...]
        cid = lax.broadcasted_iota(jnp.int32, (B, C), 1)
        tki = tki_ref[...]
        um = jnp.zeros((B, C), jnp.float32)
        for k in range(K):
            um = um + jnp.where(cid == tki[:, k:k + 1], lg[:, k:k + 1], 0.0)
        um_ref[...] = um


_zero3 = lambda i: (0, 0, 0)
_zero2 = lambda i: (0, 0)
_mega = pl.pallas_call(
    _mega_body,
    grid=(B,),
    in_specs=[
        pl.BlockSpec((B * 16, DWP), _zero2),
        pl.BlockSpec((DWP, DE), _zero2),
        pl.BlockSpec((1, DE), _zero2),
        pl.BlockSpec((DE, DT), _zero2),
        pl.BlockSpec((1, DT), _zero2),
        pl.BlockSpec(memory_space=pl.ANY),
        pl.BlockSpec((DF, DT), _zero2),
        pl.BlockSpec((DT, 1), _zero2),
        pl.BlockSpec((DT, DJ), _zero2),
        pl.BlockSpec((1, DJ), _zero2),
        pl.BlockSpec((DJ, K), _zero2),
        pl.BlockSpec((1, K), _zero2),
        pl.BlockSpec((B, K), _zero2),
    ],
    out_specs=[
        pl.BlockSpec((1, MM, K), lambda i: (i, 0, 0)),
        pl.BlockSpec((1, MM, K), lambda i: (i, 0, 0)),
        pl.BlockSpec((B, C), _zero2),
    ],
    out_shape=[
        jax.ShapeDtypeStruct((B, MM, K), jnp.float32),
        jax.ShapeDtypeStruct((B, MM, K), jnp.float32),
        jax.ShapeDtypeStruct((B, C), jnp.float32),
    ],
    scratch_shapes=[
        pltpu.VMEM((B * 16, DT), jnp.float32),
        pltpu.VMEM((B, DT), jnp.float32),
        pltpu.VMEM((2, DF, MM), jnp.float32),
        pltpu.SemaphoreType.DMA((2,)),
    ],
)


def kernel(ftm, logits, inputs, labels, indices_table, word_emb,
           W_cls, b_cls, W1, b1, W2, b2, W3, b3, Wc, bc):
    itab = indices_table.reshape(-1)
    wemb = _kpad(word_emb)
    tki_p, tkc_p, cls0 = _get_sc_call()(logits.reshape(-1), itab, wemb)
    topk_idx = tki_p.reshape(B, 16)[:, :K]
    topk_cls = tkc_p.reshape(B, 16, 4)[:, :K, :]
    wcls_p = jnp.pad(W_cls, ((0, DWP - DW), (0, 0)))
    att_w, att_sfx, um = _mega(cls0.reshape(B * 16, DWP), wcls_p,
                               b_cls[None], W1, b1[None],
                               ftm.reshape(B, DF, MM), W2, b2[:, None],
                               W3, b3[None], Wc, bc[None], topk_idx)
    return um, att_w, topk_cls, att_sfx, topk_idx, labels
